# baseline (speedup 1.0000x reference)
import functools

import jax
import jax.numpy as jnp
from jax import lax
from jax.experimental import pallas as pl
from jax.experimental.pallas import tpu as pltpu
from jax.experimental.pallas import tpu_sc as plsc

B = 8
N = 8192
K = 2048
N_SC = 2048
N_TC = N - N_SC
CHUNK = 2048
NC = N_TC // CHUNK

NW = 32
RPW = N_SC // NW
R_BLK = 16
NBLK = RPW // R_BLK


def _tc_extremes_kernel(x_ref, fe_ref, ext_out_ref, ext_ref):
    b = pl.program_id(0)
    c = pl.program_id(1)

    @pl.when(c == 0)
    def _init():
        ext_ref[b, 0] = jnp.inf
        ext_ref[b, 1] = jnp.inf
        ext_ref[b, 2] = -jnp.inf
        ext_ref[b, 3] = -jnp.inf

    vals = jax.lax.dot_general(
        fe_ref[...], x_ref[0], (((1,), (1,)), ((), ())),
        preferred_element_type=jnp.float32)

    m1 = jnp.max(vals)
    mcnt = jnp.sum(jnp.where(vals == m1, 1.0, 0.0))
    m2 = jnp.where(mcnt >= 2.0, m1,
                   jnp.max(jnp.where(vals == m1, -jnp.inf, vals)))
    n1 = jnp.min(vals)
    ncnt = jnp.sum(jnp.where(vals == n1, 1.0, 0.0))
    n2 = jnp.where(ncnt >= 2.0, n1,
                   jnp.min(jnp.where(vals == n1, jnp.inf, vals)))

    a1 = ext_ref[b, 3]
    a2 = ext_ref[b, 2]
    ext_ref[b, 3] = jnp.maximum(a1, m1)
    ext_ref[b, 2] = jnp.maximum(jnp.minimum(a1, m1), jnp.maximum(a2, m2))
    s1 = ext_ref[b, 0]
    s2 = ext_ref[b, 1]
    ext_ref[b, 0] = jnp.minimum(s1, n1)
    ext_ref[b, 1] = jnp.minimum(jnp.maximum(s1, n1), jnp.minimum(s2, n2))

    @pl.when((b == B - 1) & (c == NC - 1))
    def _emit():
        ext_out_ref[...] = jnp.stack(
            [jnp.stack([ext_ref[i, j] for j in range(4)]) for i in range(B)])


def _tc_extremes(x, fe):
    return pl.pallas_call(
        _tc_extremes_kernel,
        grid=(B, NC),
        in_specs=[
            pl.BlockSpec((1, CHUNK, K), lambda b, c: (b, c, 0)),
            pl.BlockSpec((1, K), lambda b, c: (0, 0)),
        ],
        out_specs=pl.BlockSpec((B, 4), lambda b, c: (0, 0)),
        out_shape=jax.ShapeDtypeStruct((B, 4), jnp.float32),
        scratch_shapes=[pltpu.SMEM((B, 4), jnp.float32)],
    )(x, fe)


def _set_lane(vec, lane_iota, j, val):
    return jnp.where(lane_iota == j, val, vec)


def _make_sc_extremes():
    mesh = plsc.VectorSubcoreMesh(core_axis_name="c", subcore_axis_name="s")

    GTOT = B * NBLK

    @functools.partial(
        pl.kernel, mesh=mesh,
        out_type=jax.ShapeDtypeStruct((NW, 32), jnp.float32),
        scratch_types=[
            pltpu.VMEM((R_BLK, K), jnp.float32),
            pltpu.VMEM((R_BLK, K), jnp.float32),
            pltpu.VMEM((K,), jnp.float32),
            pltpu.VMEM((32,), jnp.float32),
            pltpu.VMEM((B, 4, 16), jnp.float32),
            pltpu.SemaphoreType.DMA,
            pltpu.SemaphoreType.DMA,
        ],
    )
    def sc_extremes(x_hbm, fe_hbm, out_hbm, xbuf0, xbuf1, febuf, res_v,
                    ext_v, sem0, sem1):
        wid = lax.axis_index("s") * 2 + lax.axis_index("c")
        pltpu.sync_copy(fe_hbm, febuf)
        base = N_TC + wid * RPW
        lane_iota = lax.iota(jnp.int32, 16)
        bufs = (xbuf0, xbuf1)
        sems = (sem0, sem1)

        pinf = jnp.full((16,), jnp.inf, jnp.float32)
        ninf = jnp.full((16,), -jnp.inf, jnp.float32)
        for b in range(B):
            ext_v[b, 0] = pinf
            ext_v[b, 1] = pinf
            ext_v[b, 2] = ninf
            ext_v[b, 3] = ninf

        def _src(g):
            b = g // NBLK
            i = lax.rem(g, NBLK)
            return x_hbm.at[b, pl.ds(base + i * R_BLK, R_BLK), :]

        def _compute(xbuf, b):
            RG = 4
            KU = 4
            s1 = ext_v[b, 0]
            s2 = ext_v[b, 1]
            a2 = ext_v[b, 2]
            a1 = ext_v[b, 3]
            for r0 in range(0, R_BLK, RG):
                def k_body(k, accs):
                    accs = list(accs)
                    for dk in range(KU):
                        off = pl.multiple_of(k * (16 * KU) + dk * 16, 16)
                        f = febuf[pl.ds(off, 16)]
                        for j in range(RG):
                            accs[j] = accs[j] + xbuf[r0 + j, pl.ds(off, 16)] * f
                    return tuple(accs)
                accs = lax.fori_loop(
                    0, K // (16 * KU), k_body,
                    tuple(jnp.zeros((16,), jnp.float32) for _ in range(RG)))
                for acc in accs:
                    for d in (8, 4, 2, 1):
                        acc = acc + jnp.take(acc, lane_iota ^ d, axis=0)
                    v = acc
                    a2 = jnp.maximum(jnp.minimum(a1, v), a2)
                    a1 = jnp.maximum(a1, v)
                    s2 = jnp.minimum(jnp.maximum(s1, v), s2)
                    s1 = jnp.minimum(s1, v)
            ext_v[b, 0] = s1
            ext_v[b, 1] = s2
            ext_v[b, 2] = a2
            ext_v[b, 3] = a1

        pltpu.make_async_copy(_src(0), xbuf0, sem0).start()

        def g_body(g, carry):
            b = g // NBLK
            for par in (0, 1):
                @pl.when(lax.rem(g, 2) == par)
                def _():
                    @pl.when(g + 1 < GTOT)
                    def _():
                        pltpu.make_async_copy(
                            _src(g + 1), bufs[1 - par], sems[1 - par]).start()
                    pltpu.make_async_copy(
                        _src(g), bufs[par], sems[par]).wait()
                    _compute(bufs[par], b)
            return carry
        lax.fori_loop(0, GTOT, g_body, jnp.int32(0))

        vec0 = jnp.zeros((16,), jnp.float32)
        vec1 = jnp.zeros((16,), jnp.float32)
        for b in range(B):
            for j in range(4):
                val = ext_v[b, j]
                pos = b * 4 + j
                if pos < 16:
                    vec0 = _set_lane(vec0, lane_iota, pos, val)
                else:
                    vec1 = _set_lane(vec1, lane_iota, pos - 16, val)
        res_v[pl.ds(0, 16)] = vec0
        res_v[pl.ds(16, 16)] = vec1
        pltpu.sync_copy(res_v, out_hbm.at[wid])

    return sc_extremes


def _col_min2(col):
    mn = jnp.min(col)
    cnt = jnp.sum(jnp.where(col == mn, 1.0, 0.0))
    sec = jnp.where(cnt >= 2.0, mn,
                    jnp.min(jnp.where(col == mn, jnp.inf, col)))
    return mn, sec


def _col_max2(col):
    mx = jnp.max(col)
    cnt = jnp.sum(jnp.where(col == mx, 1.0, 0.0))
    sec = jnp.where(cnt >= 2.0, mx,
                    jnp.max(jnp.where(col == mx, -jnp.inf, col)))
    return mx, sec


def _merge_kernel(tc_ref, sc_ref, w1_ref, b1_ref, w2_ref, b2_ref,
                  w3_ref, b3_ref, out_ref):
    sc = sc_ref[...]
    rows = []
    for b in range(B):
        s1c = sc[:, 4 * b:4 * b + 1]
        s2c = sc[:, 4 * b + 1:4 * b + 2]
        a2c = sc[:, 4 * b + 2:4 * b + 3]
        a1c = sc[:, 4 * b + 3:4 * b + 4]
        g_s1, g_s1sec = _col_min2(s1c)
        g_a1, g_a1sec = _col_max2(a1c)
        g_s2 = jnp.minimum(jnp.min(s2c), g_s1sec)
        g_a2 = jnp.maximum(jnp.max(a2c), g_a1sec)
        t_s1 = tc_ref[b, 0]
        t_s2 = tc_ref[b, 1]
        t_a2 = tc_ref[b, 2]
        t_a1 = tc_ref[b, 3]
        f_s1 = jnp.minimum(t_s1, g_s1)
        f_s2 = jnp.minimum(jnp.maximum(t_s1, g_s1), jnp.minimum(t_s2, g_s2))
        f_a1 = jnp.maximum(t_a1, g_a1)
        f_a2 = jnp.maximum(jnp.minimum(t_a1, g_a1), jnp.maximum(t_a2, g_a2))
        rows.append(jnp.stack([f_s1, f_s2, f_a2, f_a1]))
    mm = jnp.stack(rows)

    h = jax.nn.sigmoid(
        jnp.dot(mm, w1_ref[...], preferred_element_type=jnp.float32)
        + b1_ref[...])
    h = jax.nn.sigmoid(
        jnp.dot(h, w2_ref[...], preferred_element_type=jnp.float32)
        + b2_ref[...])
    logits = (jnp.dot(h, w3_ref[...], preferred_element_type=jnp.float32)
              + b3_ref[...])
    z = logits - jnp.max(logits, axis=1, keepdims=True)
    e = jnp.exp(z)
    out_ref[...] = e / jnp.sum(e, axis=1, keepdims=True)


def _merge(tc_ext, sc_ext, w1t, b1r, w2t, b2r, w3t, b3r):
    full = lambda s: pl.BlockSpec(s, lambda: (0,) * len(s))
    return pl.pallas_call(
        _merge_kernel,
        in_specs=[
            full((B, 4)),
            full((NW, 32)),
            full((4, 200)),
            full((1, 200)),
            full((200, 100)),
            full((1, 100)),
            full((100, 1)),
            full((1, 1)),
        ],
        out_specs=full((B, 1)),
        out_shape=jax.ShapeDtypeStruct((B, 1), jnp.float32),
    )(tc_ext, sc_ext, w1t, b1r, w2t, b2r, w3t, b3r)


_sc_extremes = _make_sc_extremes()


def kernel(x, feature_embedding, W1, b1, W2, b2, W3, b3):
    fe_row = feature_embedding.reshape(1, K)
    w1t = W1.T
    b1r = b1.reshape(1, -1)
    w2t = W2.T
    b2r = b2.reshape(1, -1)
    w3t = W3.T
    b3r = b3.reshape(1, -1)

    tc_ext = _tc_extremes(x, fe_row)
    sc_ext = _sc_extremes(x, feature_embedding)
    return _merge(tc_ext, sc_ext, w1t, b1r, w2t, b2r, w3t, b3r)


# speedup vs baseline: 1.0063x; 1.0063x over previous
import functools

import jax
import jax.numpy as jnp
from jax import lax
from jax.experimental import pallas as pl
from jax.experimental.pallas import tpu as pltpu
from jax.experimental.pallas import tpu_sc as plsc

B = 8
N = 8192
K = 2048
N_SC = 1536
N_TC = N - N_SC
CHUNK = 1664
NC = N_TC // CHUNK

NW = 32
RPW = N_SC // NW
R_BLK = 16
NBLK = RPW // R_BLK


def _tc_extremes_kernel(x_ref, fe_ref, ext_out_ref, ext_ref):
    b = pl.program_id(0)
    c = pl.program_id(1)

    @pl.when(c == 0)
    def _init():
        ext_ref[b, 0] = jnp.inf
        ext_ref[b, 1] = jnp.inf
        ext_ref[b, 2] = -jnp.inf
        ext_ref[b, 3] = -jnp.inf

    vals = jax.lax.dot_general(
        fe_ref[...], x_ref[0], (((1,), (1,)), ((), ())),
        preferred_element_type=jnp.float32)

    m1 = jnp.max(vals)
    mcnt = jnp.sum(jnp.where(vals == m1, 1.0, 0.0))
    m2 = jnp.where(mcnt >= 2.0, m1,
                   jnp.max(jnp.where(vals == m1, -jnp.inf, vals)))
    n1 = jnp.min(vals)
    ncnt = jnp.sum(jnp.where(vals == n1, 1.0, 0.0))
    n2 = jnp.where(ncnt >= 2.0, n1,
                   jnp.min(jnp.where(vals == n1, jnp.inf, vals)))

    a1 = ext_ref[b, 3]
    a2 = ext_ref[b, 2]
    ext_ref[b, 3] = jnp.maximum(a1, m1)
    ext_ref[b, 2] = jnp.maximum(jnp.minimum(a1, m1), jnp.maximum(a2, m2))
    s1 = ext_ref[b, 0]
    s2 = ext_ref[b, 1]
    ext_ref[b, 0] = jnp.minimum(s1, n1)
    ext_ref[b, 1] = jnp.minimum(jnp.maximum(s1, n1), jnp.minimum(s2, n2))

    @pl.when((b == B - 1) & (c == NC - 1))
    def _emit():
        ext_out_ref[...] = jnp.stack(
            [jnp.stack([ext_ref[i, j] for j in range(4)]) for i in range(B)])


def _tc_extremes(x, fe):
    return pl.pallas_call(
        _tc_extremes_kernel,
        grid=(B, NC),
        in_specs=[
            pl.BlockSpec((1, CHUNK, K), lambda b, c: (b, c, 0)),
            pl.BlockSpec((1, K), lambda b, c: (0, 0)),
        ],
        out_specs=pl.BlockSpec((B, 4), lambda b, c: (0, 0)),
        out_shape=jax.ShapeDtypeStruct((B, 4), jnp.float32),
        scratch_shapes=[pltpu.SMEM((B, 4), jnp.float32)],
    )(x, fe)


def _set_lane(vec, lane_iota, j, val):
    return jnp.where(lane_iota == j, val, vec)


def _make_sc_extremes():
    mesh = plsc.VectorSubcoreMesh(core_axis_name="c", subcore_axis_name="s")

    GTOT = B * NBLK

    @functools.partial(
        pl.kernel, mesh=mesh,
        out_type=jax.ShapeDtypeStruct((NW, 32), jnp.float32),
        scratch_types=[
            pltpu.VMEM((R_BLK, K), jnp.float32),
            pltpu.VMEM((R_BLK, K), jnp.float32),
            pltpu.VMEM((K,), jnp.float32),
            pltpu.VMEM((32,), jnp.float32),
            pltpu.VMEM((B, 4, 16), jnp.float32),
            pltpu.SemaphoreType.DMA,
            pltpu.SemaphoreType.DMA,
        ],
    )
    def sc_extremes(x_hbm, fe_hbm, out_hbm, xbuf0, xbuf1, febuf, res_v,
                    ext_v, sem0, sem1):
        wid = lax.axis_index("s") * 2 + lax.axis_index("c")
        pltpu.sync_copy(fe_hbm, febuf)
        base = N_TC + wid * RPW
        lane_iota = lax.iota(jnp.int32, 16)
        bufs = (xbuf0, xbuf1)
        sems = (sem0, sem1)

        pinf = jnp.full((16,), jnp.inf, jnp.float32)
        ninf = jnp.full((16,), -jnp.inf, jnp.float32)
        for b in range(B):
            ext_v[b, 0] = pinf
            ext_v[b, 1] = pinf
            ext_v[b, 2] = ninf
            ext_v[b, 3] = ninf

        def _src(g):
            b = g // NBLK
            i = lax.rem(g, NBLK)
            return x_hbm.at[b, pl.ds(base + i * R_BLK, R_BLK), :]

        def _compute(xbuf, b):
            RG = 4
            KU = 4
            s1 = ext_v[b, 0]
            s2 = ext_v[b, 1]
            a2 = ext_v[b, 2]
            a1 = ext_v[b, 3]
            for r0 in range(0, R_BLK, RG):
                def k_body(k, accs):
                    accs = list(accs)
                    for dk in range(KU):
                        off = pl.multiple_of(k * (16 * KU) + dk * 16, 16)
                        f = febuf[pl.ds(off, 16)]
                        for j in range(RG):
                            accs[j] = accs[j] + xbuf[r0 + j, pl.ds(off, 16)] * f
                    return tuple(accs)
                accs = lax.fori_loop(
                    0, K // (16 * KU), k_body,
                    tuple(jnp.zeros((16,), jnp.float32) for _ in range(RG)))
                for acc in accs:
                    for d in (8, 4, 2, 1):
                        acc = acc + jnp.take(acc, lane_iota ^ d, axis=0)
                    v = acc
                    a2 = jnp.maximum(jnp.minimum(a1, v), a2)
                    a1 = jnp.maximum(a1, v)
                    s2 = jnp.minimum(jnp.maximum(s1, v), s2)
                    s1 = jnp.minimum(s1, v)
            ext_v[b, 0] = s1
            ext_v[b, 1] = s2
            ext_v[b, 2] = a2
            ext_v[b, 3] = a1

        pltpu.make_async_copy(_src(0), xbuf0, sem0).start()

        def g_body(g, carry):
            b = g // NBLK
            for par in (0, 1):
                @pl.when(lax.rem(g, 2) == par)
                def _():
                    @pl.when(g + 1 < GTOT)
                    def _():
                        pltpu.make_async_copy(
                            _src(g + 1), bufs[1 - par], sems[1 - par]).start()
                    pltpu.make_async_copy(
                        _src(g), bufs[par], sems[par]).wait()
                    _compute(bufs[par], b)
            return carry
        lax.fori_loop(0, GTOT, g_body, jnp.int32(0))

        vec0 = jnp.zeros((16,), jnp.float32)
        vec1 = jnp.zeros((16,), jnp.float32)
        for b in range(B):
            for j in range(4):
                val = ext_v[b, j]
                pos = b * 4 + j
                if pos < 16:
                    vec0 = _set_lane(vec0, lane_iota, pos, val)
                else:
                    vec1 = _set_lane(vec1, lane_iota, pos - 16, val)
        res_v[pl.ds(0, 16)] = vec0
        res_v[pl.ds(16, 16)] = vec1
        pltpu.sync_copy(res_v, out_hbm.at[wid])

    return sc_extremes


def _col_min2(col):
    mn = jnp.min(col)
    cnt = jnp.sum(jnp.where(col == mn, 1.0, 0.0))
    sec = jnp.where(cnt >= 2.0, mn,
                    jnp.min(jnp.where(col == mn, jnp.inf, col)))
    return mn, sec


def _col_max2(col):
    mx = jnp.max(col)
    cnt = jnp.sum(jnp.where(col == mx, 1.0, 0.0))
    sec = jnp.where(cnt >= 2.0, mx,
                    jnp.max(jnp.where(col == mx, -jnp.inf, col)))
    return mx, sec


def _merge_kernel(tc_ref, sc_ref, w1_ref, b1_ref, w2_ref, b2_ref,
                  w3_ref, b3_ref, out_ref):
    sc = sc_ref[...]
    rows = []
    for b in range(B):
        s1c = sc[:, 4 * b:4 * b + 1]
        s2c = sc[:, 4 * b + 1:4 * b + 2]
        a2c = sc[:, 4 * b + 2:4 * b + 3]
        a1c = sc[:, 4 * b + 3:4 * b + 4]
        g_s1, g_s1sec = _col_min2(s1c)
        g_a1, g_a1sec = _col_max2(a1c)
        g_s2 = jnp.minimum(jnp.min(s2c), g_s1sec)
        g_a2 = jnp.maximum(jnp.max(a2c), g_a1sec)
        t_s1 = tc_ref[b, 0]
        t_s2 = tc_ref[b, 1]
        t_a2 = tc_ref[b, 2]
        t_a1 = tc_ref[b, 3]
        f_s1 = jnp.minimum(t_s1, g_s1)
        f_s2 = jnp.minimum(jnp.maximum(t_s1, g_s1), jnp.minimum(t_s2, g_s2))
        f_a1 = jnp.maximum(t_a1, g_a1)
        f_a2 = jnp.maximum(jnp.minimum(t_a1, g_a1), jnp.maximum(t_a2, g_a2))
        rows.append(jnp.stack([f_s1, f_s2, f_a2, f_a1]))
    mm = jnp.stack(rows)

    h = jax.nn.sigmoid(
        jnp.dot(mm, w1_ref[...], preferred_element_type=jnp.float32)
        + b1_ref[...])
    h = jax.nn.sigmoid(
        jnp.dot(h, w2_ref[...], preferred_element_type=jnp.float32)
        + b2_ref[...])
    logits = (jnp.dot(h, w3_ref[...], preferred_element_type=jnp.float32)
              + b3_ref[...])
    z = logits - jnp.max(logits, axis=1, keepdims=True)
    e = jnp.exp(z)
    out_ref[...] = e / jnp.sum(e, axis=1, keepdims=True)


def _merge(tc_ext, sc_ext, w1t, b1r, w2t, b2r, w3t, b3r):
    full = lambda s: pl.BlockSpec(s, lambda: (0,) * len(s))
    return pl.pallas_call(
        _merge_kernel,
        in_specs=[
            full((B, 4)),
            full((NW, 32)),
            full((4, 200)),
            full((1, 200)),
            full((200, 100)),
            full((1, 100)),
            full((100, 1)),
            full((1, 1)),
        ],
        out_specs=full((B, 1)),
        out_shape=jax.ShapeDtypeStruct((B, 1), jnp.float32),
    )(tc_ext, sc_ext, w1t, b1r, w2t, b2r, w3t, b3r)


_sc_extremes = _make_sc_extremes()


def kernel(x, feature_embedding, W1, b1, W2, b2, W3, b3):
    fe_row = feature_embedding.reshape(1, K)
    w1t = W1.T
    b1r = b1.reshape(1, -1)
    w2t = W2.T
    b2r = b2.reshape(1, -1)
    w3t = W3.T
    b3r = b3.reshape(1, -1)

    tc_ext = _tc_extremes(x, fe_row)
    sc_ext = _sc_extremes(x, feature_embedding)
    return _merge(tc_ext, sc_ext, w1t, b1r, w2t, b2r, w3t, b3r)


# speedup vs baseline: 1.0105x; 1.0043x over previous
import functools

import jax
import jax.numpy as jnp
from jax import lax
from jax.experimental import pallas as pl
from jax.experimental.pallas import tpu as pltpu
from jax.experimental.pallas import tpu_sc as plsc

B = 8
N = 8192
K = 2048
N_SC = 1536
N_TC = N - N_SC
CHUNK = 1664
NC = N_TC // CHUNK

NW = 32
RPW = N_SC // NW
R_BLK = 16
NBLK = RPW // R_BLK


def _tc_extremes_kernel(x_ref, fe_ref, ext_out_ref, ext_ref):
    b = pl.program_id(0)
    c = pl.program_id(1)

    @pl.when(c == 0)
    def _init():
        ext_ref[b, 0] = jnp.inf
        ext_ref[b, 1] = jnp.inf
        ext_ref[b, 2] = -jnp.inf
        ext_ref[b, 3] = -jnp.inf

    vals = jax.lax.dot_general(
        fe_ref[...], x_ref[0], (((1,), (1,)), ((), ())),
        preferred_element_type=jnp.float32)

    m1 = jnp.max(vals)
    mcnt = jnp.sum(jnp.where(vals == m1, 1.0, 0.0))
    m2 = jnp.where(mcnt >= 2.0, m1,
                   jnp.max(jnp.where(vals == m1, -jnp.inf, vals)))
    n1 = jnp.min(vals)
    ncnt = jnp.sum(jnp.where(vals == n1, 1.0, 0.0))
    n2 = jnp.where(ncnt >= 2.0, n1,
                   jnp.min(jnp.where(vals == n1, jnp.inf, vals)))

    a1 = ext_ref[b, 3]
    a2 = ext_ref[b, 2]
    ext_ref[b, 3] = jnp.maximum(a1, m1)
    ext_ref[b, 2] = jnp.maximum(jnp.minimum(a1, m1), jnp.maximum(a2, m2))
    s1 = ext_ref[b, 0]
    s2 = ext_ref[b, 1]
    ext_ref[b, 0] = jnp.minimum(s1, n1)
    ext_ref[b, 1] = jnp.minimum(jnp.maximum(s1, n1), jnp.minimum(s2, n2))

    @pl.when((b == B - 1) & (c == NC - 1))
    def _emit():
        ext_out_ref[...] = jnp.stack(
            [jnp.stack([ext_ref[i, j] for j in range(4)]) for i in range(B)])


def _tc_extremes(x, fe):
    return pl.pallas_call(
        _tc_extremes_kernel,
        grid=(B, NC),
        in_specs=[
            pl.BlockSpec((1, CHUNK, K), lambda b, c: (b, c, 0)),
            pl.BlockSpec((1, K), lambda b, c: (0, 0)),
        ],
        out_specs=pl.BlockSpec((B, 4), lambda b, c: (0, 0)),
        out_shape=jax.ShapeDtypeStruct((B, 4), jnp.float32),
        scratch_shapes=[pltpu.SMEM((B, 4), jnp.float32)],
    )(x, fe)


def _set_lane(vec, lane_iota, j, val):
    return jnp.where(lane_iota == j, val, vec)


def _make_sc_extremes():
    mesh = plsc.VectorSubcoreMesh(core_axis_name="c", subcore_axis_name="s")

    GTOT = B * NBLK

    @functools.partial(
        pl.kernel, mesh=mesh,
        out_type=jax.ShapeDtypeStruct((NW, 32), jnp.float32),
        scratch_types=[
            pltpu.VMEM((R_BLK, K), jnp.float32),
            pltpu.VMEM((R_BLK, K), jnp.float32),
            pltpu.VMEM((K,), jnp.float32),
            pltpu.VMEM((32,), jnp.float32),
            pltpu.VMEM((B, 4, 16), jnp.float32),
            pltpu.SemaphoreType.DMA,
            pltpu.SemaphoreType.DMA,
        ],
    )
    def sc_extremes(x_hbm, fe_hbm, out_hbm, xbuf0, xbuf1, febuf, res_v,
                    ext_v, sem0, sem1):
        wid = lax.axis_index("s") * 2 + lax.axis_index("c")
        pltpu.sync_copy(fe_hbm, febuf)
        base = N_TC + wid * RPW
        lane_iota = lax.iota(jnp.int32, 16)
        bufs = (xbuf0, xbuf1)
        sems = (sem0, sem1)

        pinf = jnp.full((16,), jnp.inf, jnp.float32)
        ninf = jnp.full((16,), -jnp.inf, jnp.float32)
        for b in range(B):
            ext_v[b, 0] = pinf
            ext_v[b, 1] = pinf
            ext_v[b, 2] = ninf
            ext_v[b, 3] = ninf

        def _src(g):
            b = g // NBLK
            i = lax.rem(g, NBLK)
            return x_hbm.at[b, pl.ds(base + i * R_BLK, R_BLK), :]

        def _compute(xbuf, b):
            RG = 4
            KU = 4
            s1 = ext_v[b, 0]
            s2 = ext_v[b, 1]
            a2 = ext_v[b, 2]
            a1 = ext_v[b, 3]
            for r0 in range(0, R_BLK, RG):
                def k_body(k, accs):
                    accs = list(accs)
                    for dk in range(KU):
                        off = pl.multiple_of(k * (16 * KU) + dk * 16, 16)
                        f = febuf[pl.ds(off, 16)]
                        for j in range(RG):
                            idx = j * KU + dk
                            accs[idx] = accs[idx] + xbuf[r0 + j, pl.ds(off, 16)] * f
                    return tuple(accs)
                accs_flat = lax.fori_loop(
                    0, K // (16 * KU), k_body,
                    tuple(jnp.zeros((16,), jnp.float32)
                          for _ in range(RG * KU)))
                accs = []
                for j in range(RG):
                    a = accs_flat[j * KU]
                    for dk in range(1, KU):
                        a = a + accs_flat[j * KU + dk]
                    accs.append(a)
                for acc in accs:
                    for d in (8, 4, 2, 1):
                        acc = acc + jnp.take(acc, lane_iota ^ d, axis=0)
                    v = acc
                    a2 = jnp.maximum(jnp.minimum(a1, v), a2)
                    a1 = jnp.maximum(a1, v)
                    s2 = jnp.minimum(jnp.maximum(s1, v), s2)
                    s1 = jnp.minimum(s1, v)
            ext_v[b, 0] = s1
            ext_v[b, 1] = s2
            ext_v[b, 2] = a2
            ext_v[b, 3] = a1

        pltpu.make_async_copy(_src(0), xbuf0, sem0).start()

        def g_body(g, carry):
            b = g // NBLK
            for par in (0, 1):
                @pl.when(lax.rem(g, 2) == par)
                def _():
                    @pl.when(g + 1 < GTOT)
                    def _():
                        pltpu.make_async_copy(
                            _src(g + 1), bufs[1 - par], sems[1 - par]).start()
                    pltpu.make_async_copy(
                        _src(g), bufs[par], sems[par]).wait()
                    _compute(bufs[par], b)
            return carry
        lax.fori_loop(0, GTOT, g_body, jnp.int32(0))

        vec0 = jnp.zeros((16,), jnp.float32)
        vec1 = jnp.zeros((16,), jnp.float32)
        for b in range(B):
            for j in range(4):
                val = ext_v[b, j]
                pos = b * 4 + j
                if pos < 16:
                    vec0 = _set_lane(vec0, lane_iota, pos, val)
                else:
                    vec1 = _set_lane(vec1, lane_iota, pos - 16, val)
        res_v[pl.ds(0, 16)] = vec0
        res_v[pl.ds(16, 16)] = vec1
        pltpu.sync_copy(res_v, out_hbm.at[wid])

    return sc_extremes


def _col_min2(col):
    mn = jnp.min(col)
    cnt = jnp.sum(jnp.where(col == mn, 1.0, 0.0))
    sec = jnp.where(cnt >= 2.0, mn,
                    jnp.min(jnp.where(col == mn, jnp.inf, col)))
    return mn, sec


def _col_max2(col):
    mx = jnp.max(col)
    cnt = jnp.sum(jnp.where(col == mx, 1.0, 0.0))
    sec = jnp.where(cnt >= 2.0, mx,
                    jnp.max(jnp.where(col == mx, -jnp.inf, col)))
    return mx, sec


def _merge_kernel(tc_ref, sc_ref, w1_ref, b1_ref, w2_ref, b2_ref,
                  w3_ref, b3_ref, out_ref):
    sc = sc_ref[...]
    rows = []
    for b in range(B):
        s1c = sc[:, 4 * b:4 * b + 1]
        s2c = sc[:, 4 * b + 1:4 * b + 2]
        a2c = sc[:, 4 * b + 2:4 * b + 3]
        a1c = sc[:, 4 * b + 3:4 * b + 4]
        g_s1, g_s1sec = _col_min2(s1c)
        g_a1, g_a1sec = _col_max2(a1c)
        g_s2 = jnp.minimum(jnp.min(s2c), g_s1sec)
        g_a2 = jnp.maximum(jnp.max(a2c), g_a1sec)
        t_s1 = tc_ref[b, 0]
        t_s2 = tc_ref[b, 1]
        t_a2 = tc_ref[b, 2]
        t_a1 = tc_ref[b, 3]
        f_s1 = jnp.minimum(t_s1, g_s1)
        f_s2 = jnp.minimum(jnp.maximum(t_s1, g_s1), jnp.minimum(t_s2, g_s2))
        f_a1 = jnp.maximum(t_a1, g_a1)
        f_a2 = jnp.maximum(jnp.minimum(t_a1, g_a1), jnp.maximum(t_a2, g_a2))
        rows.append(jnp.stack([f_s1, f_s2, f_a2, f_a1]))
    mm = jnp.stack(rows)

    h = jax.nn.sigmoid(
        jnp.dot(mm, w1_ref[...], preferred_element_type=jnp.float32)
        + b1_ref[...])
    h = jax.nn.sigmoid(
        jnp.dot(h, w2_ref[...], preferred_element_type=jnp.float32)
        + b2_ref[...])
    logits = (jnp.dot(h, w3_ref[...], preferred_element_type=jnp.float32)
              + b3_ref[...])
    z = logits - jnp.max(logits, axis=1, keepdims=True)
    e = jnp.exp(z)
    out_ref[...] = e / jnp.sum(e, axis=1, keepdims=True)


def _merge(tc_ext, sc_ext, w1t, b1r, w2t, b2r, w3t, b3r):
    full = lambda s: pl.BlockSpec(s, lambda: (0,) * len(s))
    return pl.pallas_call(
        _merge_kernel,
        in_specs=[
            full((B, 4)),
            full((NW, 32)),
            full((4, 200)),
            full((1, 200)),
            full((200, 100)),
            full((1, 100)),
            full((100, 1)),
            full((1, 1)),
        ],
        out_specs=full((B, 1)),
        out_shape=jax.ShapeDtypeStruct((B, 1), jnp.float32),
    )(tc_ext, sc_ext, w1t, b1r, w2t, b2r, w3t, b3r)


_sc_extremes = _make_sc_extremes()


def kernel(x, feature_embedding, W1, b1, W2, b2, W3, b3):
    fe_row = feature_embedding.reshape(1, K)
    w1t = W1.T
    b1r = b1.reshape(1, -1)
    w2t = W2.T
    b2r = b2.reshape(1, -1)
    w3t = W3.T
    b3r = b3.reshape(1, -1)

    tc_ext = _tc_extremes(x, fe_row)
    sc_ext = _sc_extremes(x, feature_embedding)
    return _merge(tc_ext, sc_ext, w1t, b1r, w2t, b2r, w3t, b3r)


# speedup vs baseline: 1.1460x; 1.1340x over previous
import jax
import jax.numpy as jnp
from jax.experimental import pallas as pl
from jax.experimental.pallas import tpu as pltpu

B = 8
N = 8192
K = 2048
CHUNK = 2048
NC = N // CHUNK


def _chowder_kernel(x_ref, fe_ref, w1_ref, b1_ref, w2_ref, b2_ref,
                    w3_ref, b3_ref, out_ref, ext_ref):
    b = pl.program_id(0)
    c = pl.program_id(1)

    @pl.when(c == 0)
    def _init():
        ext_ref[b, 0] = jnp.inf
        ext_ref[b, 1] = jnp.inf
        ext_ref[b, 2] = -jnp.inf
        ext_ref[b, 3] = -jnp.inf

    vals = jax.lax.dot_general(
        fe_ref[...], x_ref[0], (((1,), (1,)), ((), ())),
        preferred_element_type=jnp.float32)

    m1 = jnp.max(vals)
    mcnt = jnp.sum(jnp.where(vals == m1, 1.0, 0.0))
    m2 = jnp.where(mcnt >= 2.0, m1,
                   jnp.max(jnp.where(vals == m1, -jnp.inf, vals)))
    n1 = jnp.min(vals)
    ncnt = jnp.sum(jnp.where(vals == n1, 1.0, 0.0))
    n2 = jnp.where(ncnt >= 2.0, n1,
                   jnp.min(jnp.where(vals == n1, jnp.inf, vals)))

    a1 = ext_ref[b, 3]
    a2 = ext_ref[b, 2]
    ext_ref[b, 3] = jnp.maximum(a1, m1)
    ext_ref[b, 2] = jnp.maximum(jnp.minimum(a1, m1), jnp.maximum(a2, m2))
    s1 = ext_ref[b, 0]
    s2 = ext_ref[b, 1]
    ext_ref[b, 0] = jnp.minimum(s1, n1)
    ext_ref[b, 1] = jnp.minimum(jnp.maximum(s1, n1), jnp.minimum(s2, n2))

    @pl.when((b == B - 1) & (c == NC - 1))
    def _head():
        mm = jnp.stack(
            [jnp.stack([ext_ref[i, j] for j in range(4)]) for i in range(B)])
        h = jax.nn.sigmoid(
            jnp.dot(mm, w1_ref[...], preferred_element_type=jnp.float32)
            + b1_ref[...])
        h = jax.nn.sigmoid(
            jnp.dot(h, w2_ref[...], preferred_element_type=jnp.float32)
            + b2_ref[...])
        logits = (jnp.dot(h, w3_ref[...], preferred_element_type=jnp.float32)
                  + b3_ref[...])
        z = logits - jnp.max(logits, axis=1, keepdims=True)
        e = jnp.exp(z)
        out_ref[...] = e / jnp.sum(e, axis=1, keepdims=True)


def kernel(x, feature_embedding, W1, b1, W2, b2, W3, b3):
    fe = feature_embedding.reshape(1, K)
    w1t = W1.T
    b1r = b1.reshape(1, -1)
    w2t = W2.T
    b2r = b2.reshape(1, -1)
    w3t = W3.T
    b3r = b3.reshape(1, -1)

    grid = (B, NC)
    out = pl.pallas_call(
        _chowder_kernel,
        grid=grid,
        in_specs=[
            pl.BlockSpec((1, CHUNK, K), lambda b, c: (b, c, 0)),
            pl.BlockSpec((1, K), lambda b, c: (0, 0)),
            pl.BlockSpec((4, 200), lambda b, c: (0, 0)),
            pl.BlockSpec((1, 200), lambda b, c: (0, 0)),
            pl.BlockSpec((200, 100), lambda b, c: (0, 0)),
            pl.BlockSpec((1, 100), lambda b, c: (0, 0)),
            pl.BlockSpec((100, 1), lambda b, c: (0, 0)),
            pl.BlockSpec((1, 1), lambda b, c: (0, 0)),
        ],
        out_specs=pl.BlockSpec((B, 1), lambda b, c: (0, 0)),
        out_shape=jax.ShapeDtypeStruct((B, 1), jnp.float32),
        scratch_shapes=[pltpu.SMEM((B, 4), jnp.float32)],
    )(x, fe, w1t, b1r, w2t, b2r, w3t, b3r)
    return out
